# baseline (speedup 1.0000x reference)
import functools

import jax
import jax.numpy as jnp
from jax import lax
from jax.experimental import pallas as pl
from jax.experimental.pallas import tpu as pltpu
from jax.experimental.pallas import tpu_sc as plsc

_N_GRID = 128
_R_S = 48
_P = 27
_CH = 128
_SIGMA_BIAS = -5.0
_NR = 192
_NPTS = 65536
_NW = 32
_PW = _NPTS // _NW
_CHK = 16
_OCHK = 128
_NOCHK = _PW // _OCHK


def _sc_interp_make():
    mesh = plsc.VectorSubcoreMesh(core_axis_name="c", subcore_axis_name="s")

    @functools.partial(
        pl.kernel,
        out_type=jax.ShapeDtypeStruct((_NR, _NPTS), jnp.float32),
        mesh=mesh,
        compiler_params=pltpu.CompilerParams(needs_layout_passes=False),
        scratch_types=[
            pltpu.VMEM((3 * _N_GRID * _NR,), jnp.float32),
            pltpu.VMEM((3 * _N_GRID,), jnp.float32),
            pltpu.VMEM((3, _PW), jnp.float32),
            pltpu.VMEM((_NR, _OCHK), jnp.float32),
            pltpu.SemaphoreType.DMA,
        ],
    )
    def sc_interp(tab_hbm, vox_hbm, xyz_hbm, out_hbm, tab_v, vox_v, x_v, o_v,
                  sem):
        wid = lax.axis_index("s") * 2 + lax.axis_index("c")
        base = wid * _PW
        pltpu.async_copy(tab_hbm, tab_v, sem).wait()
        pltpu.async_copy(vox_hbm, vox_v, sem).wait()
        pltpu.async_copy(xyz_hbm.at[:, pl.ds(base, _PW)], x_v, sem).wait()

        lane = lax.iota(jnp.int32, _CHK)

        def chunk_body(ci, carry):
            for sub in range(_OCHK // _CHK):
                lbases = []
                lerps = []
                for a in range(3):
                    x = x_v[a, pl.ds(ci * _OCHK + sub * _CHK, _CHK)]
                    av = jnp.full((_CHK,), a * _N_GRID, jnp.int32)
                    lo = jnp.zeros((_CHK,), jnp.int32)
                    for s in (64, 32, 16, 8, 4, 2, 1):
                        mid = lo + s
                        v = plsc.load_gather(vox_v, [av + mid])
                        lo = jnp.where(v < x, mid, lo)
                    vl = plsc.load_gather(vox_v, [av + lo])
                    vr = plsc.load_gather(vox_v, [av + lo + 1])
                    lerps.append((x - vl) / (vr - vl + 1e-06))
                    lbases.append(lo * _NR + a * (_N_GRID * _NR))

                col = lane + sub * _CHK

                def j_body(jb, c2):
                    j0 = jb * 6
                    for dj in range(6):
                        j = j0 + dj
                        acc = None
                        for a in range(3):
                            idx = lbases[a] + j
                            sl = plsc.load_gather(tab_v, [idx])
                            sr = plsc.load_gather(tab_v, [idx + _NR])
                            g = sl + lerps[a] * (sr - sl)
                            acc = g if acc is None else acc * g
                        plsc.store_scatter(
                            o_v, [jnp.full((_CHK,), 0, jnp.int32) + j, col],
                            acc)
                    return c2

                lax.fori_loop(0, _NR // 6, j_body, 0)
            copy = pltpu.async_copy(
                o_v, out_hbm.at[:, pl.ds(base + ci * _OCHK, _OCHK)], sem)
            copy.wait()
            return carry

        lax.fori_loop(0, _NOCHK, chunk_body, 0)

    return sc_interp


def _leaky(x):
    return jnp.where(x >= 0, x, 0.01 * x)


def _sigmoid(x):
    z = jnp.exp(-jnp.abs(x))
    return jnp.where(x >= 0, 1.0 / (1.0 + z), z / (1.0 + z))


def _softplus(x):
    return jnp.maximum(x, 0.0) + jnp.log1p(jnp.exp(-jnp.abs(x)))


def _tc_head_body(prod_ref, dirs_ref, bp_ref, asin_ref, acos_ref,
                  w2_ref, w3_ref, b1_ref, b2_ref, b3_ref, sig_ref, rgb_ref):
    blk = prod_ref.shape[1]
    prod = prod_ref[...]
    t2 = jnp.dot(bp_ref[...], prod, preferred_element_type=jnp.float32)
    sig_ref[...] = _softplus(t2[56:57] + _SIGMA_BIAS)

    d = dirs_ref[...]
    td = jnp.concatenate([d, d + d, jnp.zeros((2, blk), jnp.float32)], axis=0)
    t = jnp.concatenate([t2[0:56], td], axis=0)
    pre = (jnp.dot(asin_ref[...], jnp.sin(t), preferred_element_type=jnp.float32)
           + jnp.dot(acos_ref[...], jnp.cos(t),
                     preferred_element_type=jnp.float32)
           + b1_ref[...])
    h1 = _leaky(pre)
    h2 = _leaky(jnp.dot(w2_ref[...], h1,
                        preferred_element_type=jnp.float32) + b2_ref[...])
    rgb_ref[...] = _sigmoid(
        jnp.dot(w3_ref[...], h2, preferred_element_type=jnp.float32)
        + b3_ref[...])


_BLK = 4096


@jax.jit
def kernel(xyz, directions, voxel, sigma, feature, B, W1, b1, W2, b2, W3, b3):
    npts = xyz.shape[0]
    grid = npts // _BLK

    xyz_t = xyz.T
    dirs_t = directions.T

    stack = jnp.concatenate([feature[:, :_CH, :], sigma, feature[:, _CH:, :]],
                            axis=1)
    tab = jnp.transpose(stack, (0, 2, 1)).reshape(-1)

    prod = _sc_interp_make()(tab, voxel.reshape(-1), xyz_t)

    bt = B.T
    bp = (jnp.zeros((_CH // 2, _NR), jnp.float32)
          .at[:_P, :_CH].set(bt[:, :_CH])
          .at[:_P, _CH + _R_S:].set(bt[:, _CH:])
          .at[_P:2 * _P, :_CH].set(2.0 * bt[:, :_CH])
          .at[_P:2 * _P, _CH + _R_S:].set(2.0 * bt[:, _CH:])
          .at[56, _CH:_CH + _R_S].set(1.0))
    asin = (jnp.zeros((_CH, _CH // 2), jnp.float32)
            .at[:, :_P].set(W1[:, 0:27]).at[:, _P:2 * _P].set(W1[:, 54:81])
            .at[:, 56:59].set(W1[:, 108:111]).at[:, 59:62].set(W1[:, 114:117]))
    acos = (jnp.zeros((_CH, _CH // 2), jnp.float32)
            .at[:, :_P].set(W1[:, 27:54]).at[:, _P:2 * _P].set(W1[:, 81:108])
            .at[:, 56:59].set(W1[:, 111:114]).at[:, 59:62].set(W1[:, 117:120]))

    full = lambda *shape: pl.BlockSpec(shape, lambda i: (0,) * len(shape))
    sig, rgb = pl.pallas_call(
        _tc_head_body,
        grid=(grid,),
        in_specs=[
            pl.BlockSpec((_NR, _BLK), lambda i: (0, i)),
            pl.BlockSpec((3, _BLK), lambda i: (0, i)),
            full(_CH // 2, _NR),
            full(_CH, _CH // 2),
            full(_CH, _CH // 2),
            full(_CH, _CH),
            full(3, _CH),
            full(_CH, 1),
            full(_CH, 1),
            full(3, 1),
        ],
        out_specs=[
            pl.BlockSpec((1, _BLK), lambda i: (0, i)),
            pl.BlockSpec((3, _BLK), lambda i: (0, i)),
        ],
        out_shape=[
            jax.ShapeDtypeStruct((1, npts), jnp.float32),
            jax.ShapeDtypeStruct((3, npts), jnp.float32),
        ],
    )(prod, dirs_t, bp, asin, acos, W2, W3,
      b1[:, None], b2[:, None], b3[:, None])
    return sig[0], rgb.T


# speedup vs baseline: 3.6171x; 3.6171x over previous
import functools

import jax
import jax.numpy as jnp
from jax import lax
from jax.experimental import pallas as pl
from jax.experimental.pallas import tpu as pltpu
from jax.experimental.pallas import tpu_sc as plsc

_N_GRID = 128
_R_S = 48
_P = 27
_CH = 128
_SIGMA_BIAS = -5.0
_NR = 192
_NPTS = 65536
_NW = 32
_PW = _NPTS // _NW
_CHK = 16
_OCHK = 128
_NOCHK = _PW // _OCHK
_TSTR = 193


def _sc_interp_make():
    mesh = plsc.VectorSubcoreMesh(core_axis_name="c", subcore_axis_name="s")

    @functools.partial(
        pl.kernel,
        out_type=jax.ShapeDtypeStruct((_NR, _NPTS), jnp.float32),
        mesh=mesh,
        compiler_params=pltpu.CompilerParams(needs_layout_passes=False),
        scratch_types=[
            pltpu.VMEM((3 * _N_GRID * _TSTR,), jnp.float32),
            pltpu.VMEM((3 * _N_GRID,), jnp.float32),
            pltpu.VMEM((3, _PW), jnp.float32),
            pltpu.VMEM((_NR, _OCHK), jnp.float32),
            pltpu.SemaphoreType.DMA,
        ],
    )
    def sc_interp(tab_hbm, vox_hbm, xyz_hbm, out_hbm, tab_v, vox_v, x_v, o_v,
                  sem):
        wid = lax.axis_index("s") * 2 + lax.axis_index("c")
        base = wid * _PW
        pltpu.async_copy(tab_hbm, tab_v, sem).wait()
        pltpu.async_copy(vox_hbm, vox_v, sem).wait()
        pltpu.async_copy(xyz_hbm.at[:, pl.ds(base, _PW)], x_v, sem).wait()

        lane = lax.iota(jnp.int32, _CHK)

        def chunk_body(ci, carry):
            for sub in range(_OCHK // _CHK):
                lbases = []
                lerps = []
                for a in range(3):
                    x = x_v[a, pl.ds(ci * _OCHK + sub * _CHK, _CHK)]
                    av = jnp.full((_CHK,), a * _N_GRID, jnp.int32)
                    lo = jnp.zeros((_CHK,), jnp.int32)
                    for s in (64, 32, 16, 8, 4, 2, 1):
                        mid = lo + s
                        v = plsc.load_gather(vox_v, [av + mid])
                        lo = jnp.where(v < x, mid, lo)
                    vl = plsc.load_gather(vox_v, [av + lo])
                    vr = plsc.load_gather(vox_v, [av + lo + 1])
                    lerps.append((x - vl) / (vr - vl + 1e-06))
                    lbases.append(lo * _TSTR + a * (_N_GRID * _TSTR))

                col = lane + sub * _CHK

                def j_body(jb, c2):
                    j0 = jb * 6
                    for dj in range(6):
                        j = j0 + dj
                        acc = None
                        for a in range(3):
                            idx = lbases[a] + j
                            sl = plsc.load_gather(tab_v, [idx])
                            sr = plsc.load_gather(tab_v, [idx + _TSTR])
                            g = sl + lerps[a] * (sr - sl)
                            acc = g if acc is None else acc * g
                        plsc.store_scatter(
                            o_v, [jnp.full((_CHK,), 0, jnp.int32) + j, col],
                            acc)
                    return c2

                lax.fori_loop(0, _NR // 6, j_body, 0)
            copy = pltpu.async_copy(
                o_v, out_hbm.at[:, pl.ds(base + ci * _OCHK, _OCHK)], sem)
            copy.wait()
            return carry

        lax.fori_loop(0, _NOCHK, chunk_body, 0)

    return sc_interp


def _leaky(x):
    return jnp.where(x >= 0, x, 0.01 * x)


def _sigmoid(x):
    z = jnp.exp(-jnp.abs(x))
    return jnp.where(x >= 0, 1.0 / (1.0 + z), z / (1.0 + z))


def _softplus(x):
    return jnp.maximum(x, 0.0) + jnp.log1p(jnp.exp(-jnp.abs(x)))


def _tc_head_body(prod_ref, dirs_ref, bp_ref, asin_ref, acos_ref,
                  w2_ref, w3_ref, b1_ref, b2_ref, b3_ref, sig_ref, rgb_ref):
    blk = prod_ref.shape[1]
    prod = prod_ref[...]
    t2 = jnp.dot(bp_ref[...], prod, preferred_element_type=jnp.float32)
    sig_ref[...] = _softplus(t2[56:57] + _SIGMA_BIAS)

    d = dirs_ref[...]
    td = jnp.concatenate([d, d + d, jnp.zeros((2, blk), jnp.float32)], axis=0)
    t = jnp.concatenate([t2[0:56], td], axis=0)
    pre = (jnp.dot(asin_ref[...], jnp.sin(t), preferred_element_type=jnp.float32)
           + jnp.dot(acos_ref[...], jnp.cos(t),
                     preferred_element_type=jnp.float32)
           + b1_ref[...])
    h1 = _leaky(pre)
    h2 = _leaky(jnp.dot(w2_ref[...], h1,
                        preferred_element_type=jnp.float32) + b2_ref[...])
    rgb_ref[...] = _sigmoid(
        jnp.dot(w3_ref[...], h2, preferred_element_type=jnp.float32)
        + b3_ref[...])


_BLK = 4096


@jax.jit
def kernel(xyz, directions, voxel, sigma, feature, B, W1, b1, W2, b2, W3, b3):
    npts = xyz.shape[0]
    grid = npts // _BLK

    xyz_t = xyz.T
    dirs_t = directions.T

    stack = jnp.concatenate([feature[:, :_CH, :], sigma, feature[:, _CH:, :]],
                            axis=1)
    tab = jnp.concatenate(
        [jnp.transpose(stack, (0, 2, 1)),
         jnp.zeros((3, _N_GRID, _TSTR - _NR), jnp.float32)],
        axis=2).reshape(-1)

    prod = _sc_interp_make()(tab, voxel.reshape(-1), xyz_t)

    bt = B.T
    bp = (jnp.zeros((_CH // 2, _NR), jnp.float32)
          .at[:_P, :_CH].set(bt[:, :_CH])
          .at[:_P, _CH + _R_S:].set(bt[:, _CH:])
          .at[_P:2 * _P, :_CH].set(2.0 * bt[:, :_CH])
          .at[_P:2 * _P, _CH + _R_S:].set(2.0 * bt[:, _CH:])
          .at[56, _CH:_CH + _R_S].set(1.0))
    asin = (jnp.zeros((_CH, _CH // 2), jnp.float32)
            .at[:, :_P].set(W1[:, 0:27]).at[:, _P:2 * _P].set(W1[:, 54:81])
            .at[:, 56:59].set(W1[:, 108:111]).at[:, 59:62].set(W1[:, 114:117]))
    acos = (jnp.zeros((_CH, _CH // 2), jnp.float32)
            .at[:, :_P].set(W1[:, 27:54]).at[:, _P:2 * _P].set(W1[:, 81:108])
            .at[:, 56:59].set(W1[:, 111:114]).at[:, 59:62].set(W1[:, 117:120]))

    full = lambda *shape: pl.BlockSpec(shape, lambda i: (0,) * len(shape))
    sig, rgb = pl.pallas_call(
        _tc_head_body,
        grid=(grid,),
        in_specs=[
            pl.BlockSpec((_NR, _BLK), lambda i: (0, i)),
            pl.BlockSpec((3, _BLK), lambda i: (0, i)),
            full(_CH // 2, _NR),
            full(_CH, _CH // 2),
            full(_CH, _CH // 2),
            full(_CH, _CH),
            full(3, _CH),
            full(_CH, 1),
            full(_CH, 1),
            full(3, 1),
        ],
        out_specs=[
            pl.BlockSpec((1, _BLK), lambda i: (0, i)),
            pl.BlockSpec((3, _BLK), lambda i: (0, i)),
        ],
        out_shape=[
            jax.ShapeDtypeStruct((1, npts), jnp.float32),
            jax.ShapeDtypeStruct((3, npts), jnp.float32),
        ],
    )(prod, dirs_t, bp, asin, acos, W2, W3,
      b1[:, None], b2[:, None], b3[:, None])
    return sig[0], rgb.T


# speedup vs baseline: 10.3712x; 2.8672x over previous
import jax
import jax.numpy as jnp
from jax.experimental import pallas as pl
from jax.experimental.pallas import tpu as pltpu

_N_GRID = 128
_R_S = 48
_P = 27
_CH = 128
_SIGMA_BIAS = -5.0
_BLK = 8192
_NR = 192


def _leaky(x):
    return jnp.where(x >= 0, x, 0.01 * x)


def _sigmoid(x):
    z = jnp.exp(-jnp.abs(x))
    return jnp.where(x >= 0, 1.0 / (1.0 + z), z / (1.0 + z))


def _softplus(x):
    return jnp.maximum(x, 0.0) + jnp.log1p(jnp.exp(-jnp.abs(x)))


def _tc_body(xyz_ref, dirs_ref, voxel_ref, tl_ref, td_ref, zv_ref,
             bp_ref, asin_ref, acos_ref, w2_ref, w3_ref,
             b1_ref, b2_ref, b3_ref, sig_ref, rgb_ref):
    blk = xyz_ref.shape[1]
    prod = None
    for a in range(3):
        xa = xyz_ref[a][None, :]
        vox = voxel_ref[:, a][:, None]
        c = vox < xa
        cmp = jnp.where(c, 1.0, 0.0)
        vlr = jnp.dot(zv_ref[a], cmp, preferred_element_type=jnp.float32)
        vl = vlr[0:1]
        vr = vlr[1:2]
        lerp = (xa - vl) / (vr - vl + 1e-06)
        cmpl = jnp.where(c, jnp.broadcast_to(lerp, c.shape), 0.0)
        ga = (jnp.dot(tl_ref[a], cmp, preferred_element_type=jnp.float32)
              + jnp.dot(td_ref[a], cmpl, preferred_element_type=jnp.float32))
        prod = ga if prod is None else prod * ga

    t2 = jnp.dot(bp_ref[...], prod, preferred_element_type=jnp.float32)
    sig_ref[...] = _softplus(t2[56:57] + _SIGMA_BIAS)

    d = dirs_ref[...]
    td = jnp.concatenate([d, d + d, jnp.zeros((2, blk), jnp.float32)], axis=0)
    t = jnp.concatenate([t2[0:56], td], axis=0)
    pre = (jnp.dot(asin_ref[...], jnp.sin(t),
                   preferred_element_type=jnp.float32)
           + jnp.dot(acos_ref[...], jnp.cos(t),
                     preferred_element_type=jnp.float32)
           + b1_ref[...])
    h1 = _leaky(pre)
    h2 = _leaky(jnp.dot(w2_ref[...], h1,
                        preferred_element_type=jnp.float32) + b2_ref[...])
    rgb_ref[...] = _sigmoid(
        jnp.dot(w3_ref[...], h2, preferred_element_type=jnp.float32)
        + b3_ref[...])


def _ldiff(t):
    return jnp.concatenate([t[..., :1], t[..., 1:] - t[..., :-1]], axis=-1)


def _rdiff(t):
    z = jnp.zeros_like(t[..., :1])
    return jnp.concatenate([t[..., 1:2], t[..., 2:] - t[..., 1:-1], z], axis=-1)


@jax.jit
def kernel(xyz, directions, voxel, sigma, feature, B, W1, b1, W2, b2, W3, b3):
    npts = xyz.shape[0]
    grid = npts // _BLK

    xyz_t = xyz.T
    dirs_t = directions.T
    vox_t = voxel.T

    stack = jnp.concatenate([feature[:, :_CH, :], sigma, feature[:, _CH:, :]],
                            axis=1)
    tl = _ldiff(stack)
    td = _rdiff(stack) - tl
    vrow = voxel[:, None, :]
    zv = jnp.concatenate([_ldiff(vrow), _rdiff(vrow)], axis=1)

    bt = B.T
    bp = (jnp.zeros((_CH // 2, _NR), jnp.float32)
          .at[:_P, :_CH].set(bt[:, :_CH])
          .at[:_P, _CH + _R_S:].set(bt[:, _CH:])
          .at[_P:2 * _P, :_CH].set(2.0 * bt[:, :_CH])
          .at[_P:2 * _P, _CH + _R_S:].set(2.0 * bt[:, _CH:])
          .at[56, _CH:_CH + _R_S].set(1.0))
    asin = (jnp.zeros((_CH, _CH // 2), jnp.float32)
            .at[:, :_P].set(W1[:, 0:27]).at[:, _P:2 * _P].set(W1[:, 54:81])
            .at[:, 56:59].set(W1[:, 108:111]).at[:, 59:62].set(W1[:, 114:117]))
    acos = (jnp.zeros((_CH, _CH // 2), jnp.float32)
            .at[:, :_P].set(W1[:, 27:54]).at[:, _P:2 * _P].set(W1[:, 81:108])
            .at[:, 56:59].set(W1[:, 111:114]).at[:, 59:62].set(W1[:, 117:120]))

    full = lambda *shape: pl.BlockSpec(shape, lambda i: (0,) * len(shape))
    sig, rgb = pl.pallas_call(
        _tc_body,
        grid=(grid,),
        in_specs=[
            pl.BlockSpec((3, _BLK), lambda i: (0, i)),
            pl.BlockSpec((3, _BLK), lambda i: (0, i)),
            full(_N_GRID, 3),
            full(3, _NR, _N_GRID),
            full(3, _NR, _N_GRID),
            full(3, 2, _N_GRID),
            full(_CH // 2, _NR),
            full(_CH, _CH // 2),
            full(_CH, _CH // 2),
            full(_CH, _CH),
            full(3, _CH),
            full(_CH, 1),
            full(_CH, 1),
            full(3, 1),
        ],
        out_specs=[
            pl.BlockSpec((1, _BLK), lambda i: (0, i)),
            pl.BlockSpec((3, _BLK), lambda i: (0, i)),
        ],
        out_shape=[
            jax.ShapeDtypeStruct((1, npts), jnp.float32),
            jax.ShapeDtypeStruct((3, npts), jnp.float32),
        ],
        compiler_params=pltpu.CompilerParams(
            vmem_limit_bytes=100 * 1024 * 1024),
    )(xyz_t, dirs_t, vox_t, tl, td, zv, bp, asin, acos,
      W2, W3, b1[:, None], b2[:, None], b3[:, None])
    return sig[0], rgb.T
